# baseline (speedup 1.0000x reference)
import jax
import jax.numpy as jnp
from jax.experimental import pallas as pl
from jax.experimental.pallas import tpu as pltpu

N_NODES = 28
IN_DIM = 512
OUT_DIM = 512
HK = IN_DIM // 2


def _gcn_fused(inputs_ref, w_hbm, o_ref, b0, b1, s0, s1):
    c0 = pltpu.make_async_copy(w_hbm.at[pl.ds(0, HK), :], b0, s0)
    c0.start()
    c1 = pltpu.make_async_copy(w_hbm.at[pl.ds(HK, HK), :], b1, s1)
    c1.start()
    packed = inputs_ref[...]
    x = packed[:, :IN_DIM].astype(jnp.bfloat16)
    support = packed[:, IN_DIM:].astype(jnp.bfloat16)
    h = jnp.dot(support, x, preferred_element_type=jnp.float32)
    hb = h.astype(jnp.bfloat16)
    c0.wait()
    out = jnp.dot(hb[:, :HK], b0[...].astype(jnp.bfloat16),
                  preferred_element_type=jnp.float32)
    c1.wait()
    out = out + jnp.dot(hb[:, HK:], b1[...].astype(jnp.bfloat16),
                        preferred_element_type=jnp.float32)
    o_ref[...] = jnp.maximum(out, 0.0)


def kernel(inputs, weight):
    return pl.pallas_call(
        _gcn_fused,
        in_specs=[
            pl.BlockSpec(memory_space=pltpu.MemorySpace.VMEM),
            pl.BlockSpec(memory_space=pltpu.MemorySpace.HBM),
        ],
        out_specs=pl.BlockSpec(memory_space=pltpu.MemorySpace.VMEM),
        scratch_shapes=(
            pltpu.VMEM((HK, OUT_DIM), jnp.float32),
            pltpu.VMEM((HK, OUT_DIM), jnp.float32),
            pltpu.SemaphoreType.DMA,
            pltpu.SemaphoreType.DMA,
        ),
        out_shape=jax.ShapeDtypeStruct((N_NODES, OUT_DIM), jnp.float32),
    )(inputs, weight)


# speedup vs baseline: 1.3447x; 1.3447x over previous
import jax
import jax.numpy as jnp
from jax.experimental import pallas as pl

N_NODES = 28
IN_DIM = 512
OUT_DIM = 512
HN = OUT_DIM // 2


def _gcn_fused(inputs_ref, wa_ref, wb_ref, o_ref):
    packed = inputs_ref[...]
    x = packed[:, :IN_DIM].astype(jnp.bfloat16)
    support = packed[:, IN_DIM:].astype(jnp.bfloat16)
    h = jnp.dot(support, x, preferred_element_type=jnp.float32)
    hb = h.astype(jnp.bfloat16)
    out_a = jnp.dot(hb, wa_ref[...].astype(jnp.bfloat16),
                    preferred_element_type=jnp.float32)
    o_ref[:, :HN] = jnp.maximum(out_a, 0.0)
    out_b = jnp.dot(hb, wb_ref[...].astype(jnp.bfloat16),
                    preferred_element_type=jnp.float32)
    o_ref[:, HN:] = jnp.maximum(out_b, 0.0)


def kernel(inputs, weight):
    return pl.pallas_call(
        _gcn_fused,
        grid=(1,),
        in_specs=[
            pl.BlockSpec((N_NODES, IN_DIM + N_NODES), lambda g: (0, 0)),
            pl.BlockSpec((IN_DIM, HN), lambda g: (0, 0)),
            pl.BlockSpec((IN_DIM, HN), lambda g: (0, 1)),
        ],
        out_specs=pl.BlockSpec((N_NODES, OUT_DIM), lambda g: (0, 0)),
        out_shape=jax.ShapeDtypeStruct((N_NODES, OUT_DIM), jnp.float32),
    )(inputs, weight, weight)
